# Initial kernel scaffold; baseline (speedup 1.0000x reference)
import jax
import jax.numpy as jnp
from jax.experimental import pallas as pl


def kernel(x, edge_index, batch, W, att_src, att_dst, bias, lin_w, lin_b):
    raise NotImplementedError("write your pallas kernel here")


# speedup vs baseline: 64.8575x; 64.8575x over previous
import functools

import jax
import jax.numpy as jnp
import numpy as np
from jax import lax
from jax.experimental import pallas as pl
from jax.experimental.pallas import tpu as pltpu
from jax.experimental.pallas import tpu_sc as plsc

N = 10000
E = 320000
D = 128
H = 5
C = 10
G = 64
HC = H * C
PAD = 64
NPAD = 10240
RBLK = 1000
NBLK = N // RBLK

_HEADMAP = np.zeros((PAD,), np.int32)
for _j in range(HC):
    _HEADMAP[_j] = _j // C
for _h in range(H):
    _HEADMAP[HC + _h] = _h
for _j in range(HC + H, PAD):
    _HEADMAP[_j] = H

_EXPAND = np.zeros((8, PAD), np.float32)
for _j in range(HC + H):
    _EXPAND[_HEADMAP[_j], _j] = 1.0

_PSEL = np.zeros((PAD, PAD), np.float32)
for _j in range(HC):
    _PSEL[HC + _j // C, _j] = 1.0
    _PSEL[HC + _j // C, HC + _j // C] = 1.0

_SHIFT = np.zeros((8, PAD), np.float32)
for _h in range(H):
    _SHIFT[_h, HC + H + _h] = 1.0

_ONES_ROW = np.zeros((PAD,), np.float32)
_ONES_ROW[HC:HC + H] = 1.0


def _proj_body(x_ref, w_ref, asrc_ref, adst_ref, shift_ref,
               xp_pad_ref, adst_out_ref):
    xp = jnp.dot(x_ref[:], w_ref[:], preferred_element_type=jnp.float32)
    asrc8 = jnp.dot(xp, asrc_ref[:], preferred_element_type=jnp.float32)
    adst8 = jnp.dot(xp, adst_ref[:], preferred_element_type=jnp.float32)
    col = lax.broadcasted_iota(jnp.int32, xp.shape, 1)
    ones_part = jnp.where((col >= HC) & (col < HC + H), 1.0, 0.0)
    xp_pad_ref[:] = (xp + ones_part
                     + jnp.dot(asrc8, shift_ref[:],
                               preferred_element_type=jnp.float32))
    adst_out_ref[:] = adst8


def _run_proj(x, w_pad, a_src_m, a_dst_m, shift_m):
    return pl.pallas_call(
        _proj_body,
        grid=(NBLK,),
        in_specs=[
            pl.BlockSpec((RBLK, D), lambda i: (i, 0)),
            pl.BlockSpec((D, PAD), lambda i: (0, 0)),
            pl.BlockSpec((PAD, 8), lambda i: (0, 0)),
            pl.BlockSpec((PAD, 8), lambda i: (0, 0)),
            pl.BlockSpec((8, PAD), lambda i: (0, 0)),
        ],
        out_specs=[
            pl.BlockSpec((RBLK, PAD), lambda i: (i, 0)),
            pl.BlockSpec((RBLK, 8), lambda i: (i, 0)),
        ],
        out_shape=[
            jax.ShapeDtypeStruct((N, PAD), jnp.float32),
            jax.ShapeDtypeStruct((N, 8), jnp.float32),
        ],
    )(x, w_pad, a_src_m, a_dst_m, shift_m)


NC = 2
NS = 16
L = 16
NW = NC * NS
EPT = E // NW
EB = 80
NCHUNK = EPT // EB
NGRP = EB // L
RPT = NPAD // NS


def _lane_headmap(v):
    jj = lax.iota(jnp.int32, L) + v * L
    return jnp.where(jj < HC, jj // C, (jj - HC) % H)


def _sc_edge_body(echunks_hbm, xp_hbm, adst_hbm, zeros_hbm,
                  out_hbm, adst_v, idx_all, rows2, ex2, accum,
                  gsem0, gsem1, ssem0, ssem1):
    c = lax.axis_index("c")
    s = lax.axis_index("s")
    wid = c * NS + s
    pltpu.sync_copy(adst_hbm, adst_v)
    pltpu.sync_copy(echunks_hbm.at[wid], idx_all)
    pltpu.sync_copy(zeros_hbm, accum.at[pl.ds(s * RPT, RPT)])
    plsc.subcore_barrier()

    def compute(i, b):
        for g in range(NGRP):
            ev = lax.iota(jnp.int32, L) + g * L
            dstv = idx_all[i, 1, pl.ds(g * L, L)]
            for h in range(H):
                asrc = plsc.load_gather(
                    rows2.at[b], [ev, jnp.full((L,), HC + H + h, jnp.int32)])
                adst = plsc.load_gather(adst_v, [dstv * H + h])
                al = adst + asrc
                al = jnp.where(al > 0, al, 0.2 * al)
                plsc.store_scatter(ex2.at[b], [ev * 8 + h], jnp.exp(al))
        for e in range(EB):
            for v in range(PAD // L):
                mult = plsc.load_gather(ex2.at[b], [e * 8 + _lane_headmap(v)])
                sl = pl.ds(v * L, L)
                rows2[b, e, sl] = rows2[b, e, sl] * mult

    def pair(j, carry):
        i0 = j * 2
        i1 = i0 + 1
        d0 = pltpu.async_copy(xp_hbm.at[idx_all.at[i0, 0]], rows2.at[0], gsem0)
        d1 = pltpu.async_copy(xp_hbm.at[idx_all.at[i1, 0]], rows2.at[1], gsem1)
        d0.wait()
        compute(i0, 0)
        s0 = pltpu.async_copy(rows2.at[0], accum.at[idx_all.at[i0, 1]],
                              ssem0, add=True)
        d1.wait()
        compute(i1, 1)
        s1 = pltpu.async_copy(rows2.at[1], accum.at[idx_all.at[i1, 1]],
                              ssem1, add=True)
        s0.wait()
        s1.wait()
        return carry

    lax.fori_loop(0, NCHUNK // 2, pair, 0)
    if NCHUNK % 2:
        ilast = NCHUNK - 1
        pltpu.async_copy(xp_hbm.at[idx_all.at[ilast, 0]], rows2.at[0],
                         gsem0).wait()
        compute(ilast, 0)
        pltpu.async_copy(rows2.at[0], accum.at[idx_all.at[ilast, 1]],
                         ssem0, add=True).wait()
    plsc.subcore_barrier()
    pltpu.sync_copy(accum.at[pl.ds(s * RPT, RPT)],
                    out_hbm.at[c, pl.ds(s * RPT, RPT)])


def _run_edge_sc(echunks, xp_pad, adst_flat, zeros_blk):
    mesh = plsc.VectorSubcoreMesh(core_axis_name="c", subcore_axis_name="s",
                                  num_cores=NC, num_subcores=NS)
    return pl.kernel(
        _sc_edge_body,
        out_type=jax.ShapeDtypeStruct((NC, NPAD, PAD), jnp.float32),
        mesh=mesh,
        scratch_types=[
            pltpu.VMEM((N * H,), jnp.float32),
            pltpu.VMEM((NCHUNK, 2, EB), jnp.int32),
            pltpu.VMEM((2, EB, PAD), jnp.float32),
            pltpu.VMEM((2, EB * 8), jnp.float32),
            pltpu.VMEM_SHARED((NPAD, PAD), jnp.float32),
            pltpu.SemaphoreType.DMA,
            pltpu.SemaphoreType.DMA,
            pltpu.SemaphoreType.DMA,
            pltpu.SemaphoreType.DMA,
        ],
        compiler_params=pltpu.CompilerParams(needs_layout_passes=False,
                                             use_tc_tiling_on_sc=False),
    )(echunks, xp_pad, adst_flat, zeros_blk)


def _final_body(acc0_ref, acc1_ref, xp_ref, adst_ref, batch_ref,
                bias_ref, linw_ref, linb_ref, shiftt_ref, expand_ref,
                psel_ref, h_ref, out_ref, hsum_ref):
    i = pl.program_id(0)

    @pl.when(i == 0)
    def _():
        hsum_ref[:] = jnp.zeros_like(hsum_ref)

    xp = xp_ref[:]
    acc = acc0_ref[:] + acc1_ref[:]
    asrc8 = jnp.dot(xp, shiftt_ref[:], preferred_element_type=jnp.float32)
    alpha8 = asrc8 + adst_ref[:]
    alpha8 = jnp.where(alpha8 > 0, alpha8, 0.2 * alpha8)
    col8 = lax.broadcasted_iota(jnp.int32, alpha8.shape, 1)
    exl8 = jnp.where(col8 < H, jnp.exp(alpha8), 0.0)
    mult_loop = jnp.dot(exl8, expand_ref[:], preferred_element_type=jnp.float32)
    acc = acc + mult_loop * xp
    den = jnp.dot(acc, psel_ref[:], preferred_element_type=jnp.float32)
    col = lax.broadcasted_iota(jnp.int32, acc.shape, 1)
    xg = jnp.where(col < HC, acc / den + bias_ref[:], 0.0)
    xg = jnp.where(xg > 0, xg, jnp.exp(jnp.minimum(xg, 0.0)) - 1.0)
    xg = jnp.where(col < HC, xg, jnp.where(col == PAD - 1, 1.0, 0.0))
    gcol = lax.broadcasted_iota(jnp.int32, acc.shape, 1)
    onehot = (batch_ref[:] == gcol).astype(jnp.float32)
    hsum_ref[:] += lax.dot_general(onehot, xg, (((0,), (0,)), ((), ())),
                                   preferred_element_type=jnp.float32)

    @pl.when(i == NBLK - 1)
    def _():
        hsum = hsum_ref[:]
        cnt = jnp.maximum(hsum[:, PAD - 1:PAD], 1.0)
        hmat = hsum / cnt
        h_ref[:] = hmat[:, :HC]
        logits = jnp.dot(hmat, linw_ref[:], preferred_element_type=jnp.float32)
        out_ref[:] = jax.nn.sigmoid(logits[:, :1] + linb_ref[:])


def _run_final(acc0, acc1, xp_pad, adst_pad, batch2d, bias_pad, linw_pad,
               linb2d, shift_t, expand_m, psel_m):
    return pl.pallas_call(
        _final_body,
        grid=(NBLK,),
        in_specs=[
            pl.BlockSpec((RBLK, PAD), lambda i: (i, 0)),
            pl.BlockSpec((RBLK, PAD), lambda i: (i, 0)),
            pl.BlockSpec((RBLK, PAD), lambda i: (i, 0)),
            pl.BlockSpec((RBLK, 8), lambda i: (i, 0)),
            pl.BlockSpec((RBLK, 1), lambda i: (i, 0)),
            pl.BlockSpec((1, PAD), lambda i: (0, 0)),
            pl.BlockSpec((PAD, 8), lambda i: (0, 0)),
            pl.BlockSpec((1, 1), lambda i: (0, 0)),
            pl.BlockSpec((PAD, 8), lambda i: (0, 0)),
            pl.BlockSpec((8, PAD), lambda i: (0, 0)),
            pl.BlockSpec((PAD, PAD), lambda i: (0, 0)),
        ],
        out_specs=[
            pl.BlockSpec((G, HC), lambda i: (0, 0)),
            pl.BlockSpec((G, 1), lambda i: (0, 0)),
        ],
        out_shape=[
            jax.ShapeDtypeStruct((G, HC), jnp.float32),
            jax.ShapeDtypeStruct((G, 1), jnp.float32),
        ],
        scratch_shapes=[pltpu.VMEM((G, PAD), jnp.float32)],
    )(acc0, acc1, xp_pad, adst_pad, batch2d, bias_pad, linw_pad, linb2d,
      shift_t, expand_m, psel_m)


def kernel(x, edge_index, batch, W, att_src, att_dst, bias, lin_w, lin_b):
    w_pad = jnp.zeros((D, PAD), jnp.float32).at[:, :HC].set(W)
    a_src_m = jnp.zeros((PAD, 8), jnp.float32)
    a_src_m = a_src_m.at[:HC, :H].set(
        jax.scipy.linalg.block_diag(*[att_src[0, h][:, None] for h in range(H)]))
    a_dst_m = jnp.zeros((PAD, 8), jnp.float32)
    a_dst_m = a_dst_m.at[:HC, :H].set(
        jax.scipy.linalg.block_diag(*[att_dst[0, h][:, None] for h in range(H)]))
    bias_pad = jnp.zeros((1, PAD), jnp.float32).at[0, :HC].set(bias)
    linw_pad = jnp.zeros((PAD, 8), jnp.float32).at[:HC, 0].set(lin_w[:, 0])
    linb2d = lin_b.reshape(1, 1)
    batch2d = batch.reshape(N, 1)
    src = edge_index[0]
    dst = edge_index[1]

    shift_m = jnp.asarray(_SHIFT)
    shift_t = jnp.asarray(_SHIFT.T)
    expand_m = jnp.asarray(_EXPAND)
    psel_m = jnp.asarray(_PSEL)

    xp_pad, adst_pad = _run_proj(x, w_pad, a_src_m, a_dst_m, shift_m)
    adst_flat = adst_pad[:, :H].reshape(N * H)
    zeros_blk = jnp.zeros((RPT, PAD), jnp.float32)
    echunks = edge_index.reshape(2, NW, NCHUNK, EB).transpose(1, 2, 0, 3)
    acc = _run_edge_sc(echunks, xp_pad, adst_flat, zeros_blk)
    h, out = _run_final(acc[0], acc[1], xp_pad, adst_pad, batch2d,
                        bias_pad, linw_pad, linb2d, shift_t, expand_m, psel_m)
    return (h, out)


# speedup vs baseline: 70.8974x; 1.0931x over previous
import functools

import jax
import jax.numpy as jnp
import numpy as np
from jax import lax
from jax.experimental import pallas as pl
from jax.experimental.pallas import tpu as pltpu
from jax.experimental.pallas import tpu_sc as plsc

N = 10000
E = 320000
D = 128
H = 5
C = 10
G = 64
HC = H * C
PAD = 64
NPAD = 10240
RBLK = 1000
NBLK = N // RBLK

_HEADMAP = np.zeros((PAD,), np.int32)
for _j in range(HC):
    _HEADMAP[_j] = _j // C
for _h in range(H):
    _HEADMAP[HC + _h] = _h
for _j in range(HC + H, PAD):
    _HEADMAP[_j] = H

_EXPAND = np.zeros((8, PAD), np.float32)
for _j in range(HC + H):
    _EXPAND[_HEADMAP[_j], _j] = 1.0

_PSEL = np.zeros((PAD, PAD), np.float32)
for _j in range(HC):
    _PSEL[HC + _j // C, _j] = 1.0
    _PSEL[HC + _j // C, HC + _j // C] = 1.0

_SHIFT = np.zeros((8, PAD), np.float32)
for _h in range(H):
    _SHIFT[_h, HC + H + _h] = 1.0

_ONES_ROW = np.zeros((PAD,), np.float32)
_ONES_ROW[HC:HC + H] = 1.0


def _proj_body(x_ref, w_ref, asrc_ref, adst_ref, shift_ref,
               xp_pad_ref, adst_out_ref):
    xp = jnp.dot(x_ref[:], w_ref[:], preferred_element_type=jnp.float32)
    asrc8 = jnp.dot(xp, asrc_ref[:], preferred_element_type=jnp.float32)
    adst8 = jnp.dot(xp, adst_ref[:], preferred_element_type=jnp.float32)
    col = lax.broadcasted_iota(jnp.int32, xp.shape, 1)
    ones_part = jnp.where((col >= HC) & (col < HC + H), 1.0, 0.0)
    xp_pad_ref[:] = (xp + ones_part
                     + jnp.dot(asrc8, shift_ref[:],
                               preferred_element_type=jnp.float32))
    adst_out_ref[:] = adst8


def _run_proj(x, w_pad, a_src_m, a_dst_m, shift_m):
    return pl.pallas_call(
        _proj_body,
        grid=(NBLK,),
        in_specs=[
            pl.BlockSpec((RBLK, D), lambda i: (i, 0)),
            pl.BlockSpec((D, PAD), lambda i: (0, 0)),
            pl.BlockSpec((PAD, 8), lambda i: (0, 0)),
            pl.BlockSpec((PAD, 8), lambda i: (0, 0)),
            pl.BlockSpec((8, PAD), lambda i: (0, 0)),
        ],
        out_specs=[
            pl.BlockSpec((RBLK, PAD), lambda i: (i, 0)),
            pl.BlockSpec((RBLK, 8), lambda i: (i, 0)),
        ],
        out_shape=[
            jax.ShapeDtypeStruct((N, PAD), jnp.float32),
            jax.ShapeDtypeStruct((N, 8), jnp.float32),
        ],
    )(x, w_pad, a_src_m, a_dst_m, shift_m)


NC = 2
NS = 16
L = 16
NW = NC * NS
EPT = E // NW
EB = 80
NCHUNK = EPT // EB
NGRP = EB // L
RPT = NPAD // NS


def _lane_headmap(v):
    jj = lax.iota(jnp.int32, L) + v * L
    return jnp.where(jj < HC, jj // C, (jj - HC) % H)


def _sc_edge_body(echunks_hbm, xp_hbm, adst_hbm, zeros_hbm,
                  out_hbm, adst_v, idx_all, rows2, ex2, accum,
                  gsem0, gsem1, ssem0, ssem1):
    c = lax.axis_index("c")
    s = lax.axis_index("s")
    wid = c * NS + s
    pltpu.sync_copy(adst_hbm, adst_v)
    pltpu.sync_copy(echunks_hbm.at[wid], idx_all)
    pltpu.sync_copy(zeros_hbm, accum.at[pl.ds(s * RPT, RPT)])
    plsc.subcore_barrier()

    def compute(i, b):
        for g in range(NGRP):
            ev = lax.iota(jnp.int32, L) + g * L
            dstv = idx_all[i, 1, pl.ds(g * L, L)]
            for h in range(H):
                asrc = plsc.load_gather(
                    rows2.at[b], [ev, jnp.full((L,), HC + H + h, jnp.int32)])
                adst = plsc.load_gather(adst_v, [dstv * H + h])
                al = adst + asrc
                al = jnp.where(al > 0, al, 0.2 * al)
                plsc.store_scatter(ex2.at[b], [ev * 8 + h], jnp.exp(al))
        for e in range(EB):
            for v in range(PAD // L):
                mult = plsc.load_gather(ex2.at[b], [e * 8 + _lane_headmap(v)])
                sl = pl.ds(v * L, L)
                rows2[b, e, sl] = rows2[b, e, sl] * mult

    gsem = (gsem0, gsem1)
    ssem = (ssem0, ssem1)

    pltpu.async_copy(xp_hbm.at[idx_all.at[0, 0]], rows2.at[0], gsem0)

    def step(j, carry):
        def phase(b, nb):
            pltpu.make_async_copy(xp_hbm.at[idx_all.at[j, 0]],
                                  rows2.at[b], gsem[b]).wait()

            @pl.when(j >= 1)
            def _():
                pltpu.make_async_copy(
                    rows2.at[nb], accum.at[idx_all.at[j - 1, 1]],
                    ssem[nb]).wait()

            @pl.when(j + 1 < NCHUNK)
            def _():
                pltpu.async_copy(xp_hbm.at[idx_all.at[j + 1, 0]],
                                 rows2.at[nb], gsem[nb])

            compute(j, b)
            pltpu.async_copy(rows2.at[b], accum.at[idx_all.at[j, 1]],
                             ssem[b], add=True)

        @pl.when(j % 2 == 0)
        def _():
            phase(0, 1)

        @pl.when(j % 2 == 1)
        def _():
            phase(1, 0)

        return carry

    lax.fori_loop(0, NCHUNK, step, 0)
    pltpu.make_async_copy(rows2.at[(NCHUNK - 1) % 2],
                          accum.at[idx_all.at[NCHUNK - 1, 1]],
                          ssem[(NCHUNK - 1) % 2]).wait()
    plsc.subcore_barrier()
    pltpu.sync_copy(accum.at[pl.ds(s * RPT, RPT)],
                    out_hbm.at[c, pl.ds(s * RPT, RPT)])


def _run_edge_sc(echunks, xp_pad, adst_flat, zeros_blk):
    mesh = plsc.VectorSubcoreMesh(core_axis_name="c", subcore_axis_name="s",
                                  num_cores=NC, num_subcores=NS)
    return pl.kernel(
        _sc_edge_body,
        out_type=jax.ShapeDtypeStruct((NC, NPAD, PAD), jnp.float32),
        mesh=mesh,
        scratch_types=[
            pltpu.VMEM((N * H,), jnp.float32),
            pltpu.VMEM((NCHUNK, 2, EB), jnp.int32),
            pltpu.VMEM((2, EB, PAD), jnp.float32),
            pltpu.VMEM((2, EB * 8), jnp.float32),
            pltpu.VMEM_SHARED((NPAD, PAD), jnp.float32),
            pltpu.SemaphoreType.DMA,
            pltpu.SemaphoreType.DMA,
            pltpu.SemaphoreType.DMA,
            pltpu.SemaphoreType.DMA,
        ],
        compiler_params=pltpu.CompilerParams(needs_layout_passes=False,
                                             use_tc_tiling_on_sc=False),
    )(echunks, xp_pad, adst_flat, zeros_blk)


def _final_body(acc0_ref, acc1_ref, xp_ref, adst_ref, batch_ref,
                bias_ref, linw_ref, linb_ref, shiftt_ref, expand_ref,
                psel_ref, h_ref, out_ref, hsum_ref):
    i = pl.program_id(0)

    @pl.when(i == 0)
    def _():
        hsum_ref[:] = jnp.zeros_like(hsum_ref)

    xp = xp_ref[:]
    acc = acc0_ref[:] + acc1_ref[:]
    asrc8 = jnp.dot(xp, shiftt_ref[:], preferred_element_type=jnp.float32)
    alpha8 = asrc8 + adst_ref[:]
    alpha8 = jnp.where(alpha8 > 0, alpha8, 0.2 * alpha8)
    col8 = lax.broadcasted_iota(jnp.int32, alpha8.shape, 1)
    exl8 = jnp.where(col8 < H, jnp.exp(alpha8), 0.0)
    mult_loop = jnp.dot(exl8, expand_ref[:], preferred_element_type=jnp.float32)
    acc = acc + mult_loop * xp
    den = jnp.dot(acc, psel_ref[:], preferred_element_type=jnp.float32)
    col = lax.broadcasted_iota(jnp.int32, acc.shape, 1)
    xg = jnp.where(col < HC, acc / den + bias_ref[:], 0.0)
    xg = jnp.where(xg > 0, xg, jnp.exp(jnp.minimum(xg, 0.0)) - 1.0)
    xg = jnp.where(col < HC, xg, jnp.where(col == PAD - 1, 1.0, 0.0))
    gcol = lax.broadcasted_iota(jnp.int32, acc.shape, 1)
    onehot = (batch_ref[:] == gcol).astype(jnp.float32)
    hsum_ref[:] += lax.dot_general(onehot, xg, (((0,), (0,)), ((), ())),
                                   preferred_element_type=jnp.float32)

    @pl.when(i == NBLK - 1)
    def _():
        hsum = hsum_ref[:]
        cnt = jnp.maximum(hsum[:, PAD - 1:PAD], 1.0)
        hmat = hsum / cnt
        h_ref[:] = hmat[:, :HC]
        logits = jnp.dot(hmat, linw_ref[:], preferred_element_type=jnp.float32)
        out_ref[:] = jax.nn.sigmoid(logits[:, :1] + linb_ref[:])


def _run_final(acc0, acc1, xp_pad, adst_pad, batch2d, bias_pad, linw_pad,
               linb2d, shift_t, expand_m, psel_m):
    return pl.pallas_call(
        _final_body,
        grid=(NBLK,),
        in_specs=[
            pl.BlockSpec((RBLK, PAD), lambda i: (i, 0)),
            pl.BlockSpec((RBLK, PAD), lambda i: (i, 0)),
            pl.BlockSpec((RBLK, PAD), lambda i: (i, 0)),
            pl.BlockSpec((RBLK, 8), lambda i: (i, 0)),
            pl.BlockSpec((RBLK, 1), lambda i: (i, 0)),
            pl.BlockSpec((1, PAD), lambda i: (0, 0)),
            pl.BlockSpec((PAD, 8), lambda i: (0, 0)),
            pl.BlockSpec((1, 1), lambda i: (0, 0)),
            pl.BlockSpec((PAD, 8), lambda i: (0, 0)),
            pl.BlockSpec((8, PAD), lambda i: (0, 0)),
            pl.BlockSpec((PAD, PAD), lambda i: (0, 0)),
        ],
        out_specs=[
            pl.BlockSpec((G, HC), lambda i: (0, 0)),
            pl.BlockSpec((G, 1), lambda i: (0, 0)),
        ],
        out_shape=[
            jax.ShapeDtypeStruct((G, HC), jnp.float32),
            jax.ShapeDtypeStruct((G, 1), jnp.float32),
        ],
        scratch_shapes=[pltpu.VMEM((G, PAD), jnp.float32)],
    )(acc0, acc1, xp_pad, adst_pad, batch2d, bias_pad, linw_pad, linb2d,
      shift_t, expand_m, psel_m)


def kernel(x, edge_index, batch, W, att_src, att_dst, bias, lin_w, lin_b):
    w_pad = jnp.zeros((D, PAD), jnp.float32).at[:, :HC].set(W)
    a_src_m = jnp.zeros((PAD, 8), jnp.float32)
    a_src_m = a_src_m.at[:HC, :H].set(
        jax.scipy.linalg.block_diag(*[att_src[0, h][:, None] for h in range(H)]))
    a_dst_m = jnp.zeros((PAD, 8), jnp.float32)
    a_dst_m = a_dst_m.at[:HC, :H].set(
        jax.scipy.linalg.block_diag(*[att_dst[0, h][:, None] for h in range(H)]))
    bias_pad = jnp.zeros((1, PAD), jnp.float32).at[0, :HC].set(bias)
    linw_pad = jnp.zeros((PAD, 8), jnp.float32).at[:HC, 0].set(lin_w[:, 0])
    linb2d = lin_b.reshape(1, 1)
    batch2d = batch.reshape(N, 1)
    src = edge_index[0]
    dst = edge_index[1]

    shift_m = jnp.asarray(_SHIFT)
    shift_t = jnp.asarray(_SHIFT.T)
    expand_m = jnp.asarray(_EXPAND)
    psel_m = jnp.asarray(_PSEL)

    xp_pad, adst_pad = _run_proj(x, w_pad, a_src_m, a_dst_m, shift_m)
    adst_flat = adst_pad[:, :H].reshape(N * H)
    zeros_blk = jnp.zeros((RPT, PAD), jnp.float32)
    echunks = edge_index.reshape(2, NW, NCHUNK, EB).transpose(1, 2, 0, 3)
    acc = _run_edge_sc(echunks, xp_pad, adst_flat, zeros_blk)
    h, out = _run_final(acc[0], acc[1], xp_pad, adst_pad, batch2d,
                        bias_pad, linw_pad, linb2d, shift_t, expand_m, psel_m)
    return (h, out)
